# Initial kernel scaffold; baseline (speedup 1.0000x reference)
import jax
import jax.numpy as jnp
from jax.experimental import pallas as pl


def kernel(h, e, edge_index):
    raise NotImplementedError("write your pallas kernel here")


# speedup vs baseline: 1.0640x; 1.0640x over previous
import functools

import jax
import jax.numpy as jnp
from jax import lax
from jax.experimental import pallas as pl
from jax.experimental.pallas import tpu as pltpu
from jax.experimental.pallas import tpu_sc as plsc

N = 10000
E = 160000
D = 256
NCHUNK = 8
CW = 32
NROUND = NCHUNK // 2
K = 80
EPT = E // 16
NBLK = EPT // K
NPT = N // 16
FSUB = 125
NFS = NPT // FSUB
L = 16


def _sc_grid_kernel(h_all, h3, e3, src, dst, out,
                    srcb, dstb, src2b, dst2b, eb, hsb, hdb,
                    updf, updb, accfb, accbb, hb, hhb, zb,
                    accf, accb, sem1, sem2):
    cid = lax.axis_index("c")
    sid = lax.axis_index("s")

    def zinit(j, _):
        z = jnp.zeros((L,), jnp.float32)
        for cc in range(2):
            for q in range(2):
                zb[j, cc, pl.ds(q * L, L)] = z
        return 0
    lax.fori_loop(0, FSUB, zinit, 0)

    for r in range(NROUND):
        chunk = 2 * r + cid

        if r == 0:
            for k in range(NFS):
                node0 = sid * NPT + k * FSUB
                pltpu.sync_copy(zb, accf.at[pl.ds(node0, FSUB)])
                pltpu.sync_copy(zb, accb.at[pl.ds(node0, FSUB)])
            plsc.subcore_barrier()

        def edge_blk(blk, _):
            base = sid * EPT + blk * K
            pltpu.sync_copy(src.at[pl.ds(base, K)], srcb)
            pltpu.sync_copy(dst.at[pl.ds(base, K)], dstb)
            pltpu.sync_copy(e3.at[pl.ds(base, K), chunk, :], eb)

            def mkidx(i, _):
                s = srcb[pl.ds(i * L, L)]
                d = dstb[pl.ds(i * L, L)]
                src2b[pl.ds(i * L, L)] = s * NCHUNK + chunk
                dst2b[pl.ds(i * L, L)] = d * NCHUNK + chunk
                return 0
            lax.fori_loop(0, K // L, mkidx, 0)

            cp1 = pltpu.async_copy(h_all.at[src2b], hsb, sem1)
            cp2 = pltpu.async_copy(h_all.at[dst2b], hdb, sem2)
            cp1.wait()
            cp2.wait()

            def compute(j, _):
                for q in range(CW // L):
                    sl = pl.ds(q * L, L)
                    hs = hsb[j, sl]
                    hd = hdb[j, sl]
                    ev = eb[j, sl]
                    x = hs * 0.1 + ev - hd * 10.0
                    sig = 1.0 / (1.0 + jnp.exp(-x))
                    updf[j, 0, sl] = sig * (hs * 100.0)
                    updf[j, 1, sl] = sig
                    updb[j, 0, sl] = -(sig * hd)
                    updb[j, 1, sl] = sig
                return 0
            lax.fori_loop(0, K, compute, 0)

            pltpu.sync_copy(updf, accf.at[dstb], add=True)
            pltpu.sync_copy(updb, accb.at[srcb], add=True)
            return 0
        lax.fori_loop(0, NBLK, edge_blk, 0)

        plsc.subcore_barrier()

        for k in range(NFS):
            node0 = sid * NPT + k * FSUB
            pltpu.sync_copy(accf.at[pl.ds(node0, FSUB)], accfb)
            pltpu.sync_copy(accb.at[pl.ds(node0, FSUB)], accbb)
            pltpu.sync_copy(h3.at[pl.ds(node0, FSUB), chunk, :], hb)

            def fl(j, _):
                for q in range(CW // L):
                    sl = pl.ds(q * L, L)
                    nf = accfb[j, 0, sl]
                    df = accfb[j, 1, sl]
                    nb = accbb[j, 0, sl]
                    db = accbb[j, 1, sl]
                    hv = hb[j, sl]
                    hhb[j, sl] = (hv * 10.0 + nf / (df + 1e-6)
                                  + nb / (db + 1e-6))
                return 0
            lax.fori_loop(0, FSUB, fl, 0)

            pltpu.sync_copy(hhb, out.at[chunk, pl.ds(node0, FSUB), :])
            pltpu.sync_copy(zb, accf.at[pl.ds(node0, FSUB)])
            pltpu.sync_copy(zb, accb.at[pl.ds(node0, FSUB)])

        plsc.subcore_barrier()


def _make_sc_kernel():
    mesh = plsc.VectorSubcoreMesh(core_axis_name="c", subcore_axis_name="s")
    return functools.partial(
        pl.kernel,
        mesh=mesh,
        compiler_params=pltpu.CompilerParams(use_tc_tiling_on_sc=False),
        out_type=jax.ShapeDtypeStruct((NCHUNK, N, CW), jnp.float32),
        scratch_types=[
            pltpu.VMEM((K,), jnp.int32),
            pltpu.VMEM((K,), jnp.int32),
            pltpu.VMEM((K,), jnp.int32),
            pltpu.VMEM((K,), jnp.int32),
            pltpu.VMEM((K, CW), jnp.float32),
            pltpu.VMEM((K, CW), jnp.float32),
            pltpu.VMEM((K, CW), jnp.float32),
            pltpu.VMEM((K, 2, CW), jnp.float32),
            pltpu.VMEM((K, 2, CW), jnp.float32),
            pltpu.VMEM((FSUB, 2, CW), jnp.float32),
            pltpu.VMEM((FSUB, 2, CW), jnp.float32),
            pltpu.VMEM((FSUB, CW), jnp.float32),
            pltpu.VMEM((FSUB, CW), jnp.float32),
            pltpu.VMEM((FSUB, 2, CW), jnp.float32),
            pltpu.VMEM_SHARED((N, 2, CW), jnp.float32),
            pltpu.VMEM_SHARED((N, 2, CW), jnp.float32),
            pltpu.SemaphoreType.DMA,
            pltpu.SemaphoreType.DMA,
        ],
    )(_sc_grid_kernel)


def _epilogue_body(hh_ref, h_ref, out_ref):
    x = hh_ref[0]
    mean = jnp.mean(x, axis=0, keepdims=True)
    xc = x - mean
    var = jnp.mean(xc * xc, axis=0, keepdims=True)
    y = xc * lax.rsqrt(var + 1e-5)
    out_ref[0] = jnp.maximum(y, 0.0) + h_ref[0]


def kernel(h, e, edge_index):
    h = h.astype(jnp.float32)
    e = e.astype(jnp.float32)
    src = edge_index[0].astype(jnp.int32)
    dst = edge_index[1].astype(jnp.int32)

    h_all = h.reshape(N * NCHUNK, CW)
    h3 = h.reshape(N, NCHUNK, CW)
    e3 = e.reshape(E, NCHUNK, CW)

    sc = _make_sc_kernel()
    hh_pre = sc(h_all, h3, e3, src, dst)

    h_r = h3.transpose(1, 0, 2)
    out_r = pl.pallas_call(
        _epilogue_body,
        grid=(NCHUNK,),
        in_specs=[
            pl.BlockSpec((1, N, CW), lambda c: (c, 0, 0)),
            pl.BlockSpec((1, N, CW), lambda c: (c, 0, 0)),
        ],
        out_specs=pl.BlockSpec((1, N, CW), lambda c: (c, 0, 0)),
        out_shape=jax.ShapeDtypeStruct((NCHUNK, N, CW), jnp.float32),
    )(hh_pre, h_r)

    return out_r.transpose(1, 0, 2).reshape(N, D)


# speedup vs baseline: 1.3780x; 1.2951x over previous
import functools

import jax
import jax.numpy as jnp
from jax import lax
from jax.experimental import pallas as pl
from jax.experimental.pallas import tpu as pltpu
from jax.experimental.pallas import tpu_sc as plsc

N = 10000
E = 160000
D = 256
NCHUNK = 8
CW = 32
NROUND = NCHUNK // 2
K = 80
EPT = E // 16
NBLK = EPT // K
NPT = N // 16
FSUB = 125
NFS = NPT // FSUB
L = 16


def _sc_grid_kernel(h_all, h3, e3, src, dst, out,
                    srcb, dstb, src2b, dst2b, eb0, eb1, hsb, hdb,
                    updf, updb,
                    srcS0, srcS1, dstS0, dstS1,
                    accfb, accbb, hb, zb,
                    accf, accb, isem, g1, g2, s1, s2):
    ebs = (eb0, eb1)
    srcSs = (srcS0, srcS1)
    dstSs = (dstS0, dstS1)
    cid = lax.axis_index("c")
    sid = lax.axis_index("s")

    def zinit(j, _):
        z = jnp.zeros((L,), jnp.float32)
        for cc in range(2):
            for q in range(2):
                zb[j, cc, pl.ds(q * L, L)] = z
        return 0
    lax.fori_loop(0, FSUB, zinit, 0)

    for r in range(NROUND):
        chunk = 2 * r + cid

        if r == 0:
            for k in range(NFS):
                node0 = sid * NPT + k * FSUB
                pltpu.sync_copy(zb, accf.at[pl.ds(node0, FSUB)])
                pltpu.sync_copy(zb, accb.at[pl.ds(node0, FSUB)])
            plsc.subcore_barrier()

        def start_in(blk, p):
            base = sid * EPT + blk * K
            pltpu.async_copy(src.at[pl.ds(base, K)], srcb, isem)
            pltpu.async_copy(dst.at[pl.ds(base, K)], dstb, isem)
            pltpu.async_copy(e3.at[pl.ds(base, K), chunk, :], ebs[p], isem)

        def wait_in(blk, p):
            base = sid * EPT + blk * K
            pltpu.make_async_copy(src.at[pl.ds(base, K)], srcb, isem).wait()
            pltpu.make_async_copy(dst.at[pl.ds(base, K)], dstb, isem).wait()
            pltpu.make_async_copy(
                e3.at[pl.ds(base, K), chunk, :], ebs[p], isem).wait()

        def wait_scat(p):
            pltpu.make_async_copy(updf, accf.at[dstSs[p]], s1).wait()
            pltpu.make_async_copy(updb, accb.at[srcSs[p]], s2).wait()

        def body(blk, p, warm, has_next):
            wait_in(blk, p)

            def mkidx(i, _):
                s = srcb[pl.ds(i * L, L)]
                d = dstb[pl.ds(i * L, L)]
                src2b[pl.ds(i * L, L)] = s * NCHUNK + chunk
                dst2b[pl.ds(i * L, L)] = d * NCHUNK + chunk
                srcSs[p][pl.ds(i * L, L)] = s
                dstSs[p][pl.ds(i * L, L)] = d
                return 0
            lax.fori_loop(0, K // L, mkidx, 0)

            if has_next:
                start_in(blk + 1, p ^ 1)
            cp1 = pltpu.async_copy(h_all.at[src2b], hsb, g1)
            cp2 = pltpu.async_copy(h_all.at[dst2b], hdb, g2)
            if warm:
                wait_scat(p ^ 1)
            cp1.wait()
            cp2.wait()

            eb = ebs[p]

            def compute(j, _):
                for q in range(CW // L):
                    sl = pl.ds(q * L, L)
                    hs = hsb[j, sl]
                    hd = hdb[j, sl]
                    ev = eb[j, sl]
                    x = hs * 0.1 + ev - hd * 10.0
                    sig = 1.0 / (1.0 + jnp.exp(-x))
                    updf[j, 0, sl] = sig * (hs * 100.0)
                    updf[j, 1, sl] = sig
                    updb[j, 0, sl] = -(sig * hd)
                    updb[j, 1, sl] = sig
                return 0
            lax.fori_loop(0, K, compute, 0)

            pltpu.async_copy(updf, accf.at[dstSs[p]], s1, add=True)
            pltpu.async_copy(updb, accb.at[srcSs[p]], s2, add=True)

        start_in(0, 0)
        body(0, 0, warm=False, has_next=True)
        body(1, 1, warm=True, has_next=True)

        def pair(i, _):
            blk = 2 + 2 * i
            body(blk, 0, warm=True, has_next=True)
            body(blk + 1, 1, warm=True, has_next=True)
            return 0
        lax.fori_loop(0, (NBLK - 3) // 2, pair, 0)

        body(NBLK - 1, 0, warm=True, has_next=False)
        wait_scat(0)

        plsc.subcore_barrier()

        for k in range(NFS):
            node0 = sid * NPT + k * FSUB
            pltpu.sync_copy(accf.at[pl.ds(node0, FSUB)], accfb)
            pltpu.sync_copy(accb.at[pl.ds(node0, FSUB)], accbb)
            pltpu.sync_copy(h3.at[pl.ds(node0, FSUB), chunk, :], hb)

            def fl(j, _):
                for q in range(CW // L):
                    sl = pl.ds(q * L, L)
                    nf = accfb[j, 0, sl]
                    df = accfb[j, 1, sl]
                    nb = accbb[j, 0, sl]
                    db = accbb[j, 1, sl]
                    hv = hb[j, sl]
                    hb[j, sl] = (hv * 10.0 + nf / (df + 1e-6)
                                 + nb / (db + 1e-6))
                return 0
            lax.fori_loop(0, FSUB, fl, 0)

            pltpu.sync_copy(hb, out.at[pl.ds(node0, FSUB), chunk, :])
            pltpu.sync_copy(zb, accf.at[pl.ds(node0, FSUB)])
            pltpu.sync_copy(zb, accb.at[pl.ds(node0, FSUB)])

        plsc.subcore_barrier()


def _make_sc_kernel():
    mesh = plsc.VectorSubcoreMesh(core_axis_name="c", subcore_axis_name="s")
    return functools.partial(
        pl.kernel,
        mesh=mesh,
        compiler_params=pltpu.CompilerParams(use_tc_tiling_on_sc=False),
        out_type=jax.ShapeDtypeStruct((N, NCHUNK, CW), jnp.float32),
        scratch_types=[
            pltpu.VMEM((K,), jnp.int32),
            pltpu.VMEM((K,), jnp.int32),
            pltpu.VMEM((K,), jnp.int32),
            pltpu.VMEM((K,), jnp.int32),
            pltpu.VMEM((K, CW), jnp.float32),
            pltpu.VMEM((K, CW), jnp.float32),
            pltpu.VMEM((K, CW), jnp.float32),
            pltpu.VMEM((K, CW), jnp.float32),
            pltpu.VMEM((K, 2, CW), jnp.float32),
            pltpu.VMEM((K, 2, CW), jnp.float32),
            pltpu.VMEM((K,), jnp.int32),
            pltpu.VMEM((K,), jnp.int32),
            pltpu.VMEM((K,), jnp.int32),
            pltpu.VMEM((K,), jnp.int32),
            pltpu.VMEM((FSUB, 2, CW), jnp.float32),
            pltpu.VMEM((FSUB, 2, CW), jnp.float32),
            pltpu.VMEM((FSUB, CW), jnp.float32),
            pltpu.VMEM((FSUB, 2, CW), jnp.float32),
            pltpu.VMEM_SHARED((N, 2, CW), jnp.float32),
            pltpu.VMEM_SHARED((N, 2, CW), jnp.float32),
            pltpu.SemaphoreType.DMA,
            pltpu.SemaphoreType.DMA,
            pltpu.SemaphoreType.DMA,
            pltpu.SemaphoreType.DMA,
            pltpu.SemaphoreType.DMA,
        ],
    )(_sc_grid_kernel)


def _epilogue_body(hh_ref, h_ref, out_ref):
    x = hh_ref[...]
    mean = jnp.mean(x, axis=0, keepdims=True)
    xc = x - mean
    var = jnp.mean(xc * xc, axis=0, keepdims=True)
    y = xc * lax.rsqrt(var + 1e-5)
    out_ref[...] = jnp.maximum(y, 0.0) + h_ref[...]


def kernel(h, e, edge_index):
    h = h.astype(jnp.float32)
    e = e.astype(jnp.float32)
    src = edge_index[0].astype(jnp.int32)
    dst = edge_index[1].astype(jnp.int32)

    h_all = h.reshape(N * NCHUNK, CW)
    h3 = h.reshape(N, NCHUNK, CW)
    e3 = e.reshape(E, NCHUNK, CW)

    sc = _make_sc_kernel()
    hh_pre = sc(h_all, h3, e3, src, dst).reshape(N, D)

    CB = 128
    return pl.pallas_call(
        _epilogue_body,
        grid=(D // CB,),
        in_specs=[
            pl.BlockSpec((N, CB), lambda c: (0, c)),
            pl.BlockSpec((N, CB), lambda c: (0, c)),
        ],
        out_specs=pl.BlockSpec((N, CB), lambda c: (0, c)),
        out_shape=jax.ShapeDtypeStruct((N, D), jnp.float32),
    )(hh_pre, h)


# speedup vs baseline: 1.8508x; 1.3431x over previous
import functools

import jax
import jax.numpy as jnp
from jax import lax
from jax.experimental import pallas as pl
from jax.experimental.pallas import tpu as pltpu
from jax.experimental.pallas import tpu_sc as plsc

N = 10000
E = 160000
D = 256
NCHUNK = 8
CW = 32
NROUND = NCHUNK // 2
K = 80
EPT = E // 16
NBLK = EPT // K
NPT = N // 16
FSUB = 125
NFS = NPT // FSUB
L = 16
UNROLL = 4


def _sc_grid_kernel(h_all, h3, e_all, src, dst, out,
                    srcb0, srcb1, dstb0, dstb1,
                    src2b0, src2b1, dst2b0, dst2b1, eidx0, eidx1,
                    eb0, eb1, hsb0, hsb1, hdb0, hdb1,
                    updf, updb, srcS0, srcS1, dstS0, dstS1,
                    accfb, accbb, hb,
                    accf, accb, isem, g1, g2, g3, s1, s2):
    srcbs = (srcb0, srcb1)
    dstbs = (dstb0, dstb1)
    src2bs = (src2b0, src2b1)
    dst2bs = (dst2b0, dst2b1)
    eidxs = (eidx0, eidx1)
    ebs = (eb0, eb1)
    hsbs = (hsb0, hsb1)
    hdbs = (hdb0, hdb1)
    srcSs = (srcS0, srcS1)
    dstSs = (dstS0, dstS1)

    cid = lax.axis_index("c")
    sid = lax.axis_index("s")

    def z0(j, _):
        z = jnp.zeros((L,), jnp.float32)
        for cc in range(2):
            for q in range(CW // L):
                accfb[j, cc, pl.ds(q * L, L)] = z
        return 0
    lax.fori_loop(0, FSUB, z0, 0)
    for k in range(NFS):
        node0 = sid * NPT + k * FSUB
        pltpu.sync_copy(accfb, accf.at[pl.ds(node0, FSUB)])
        pltpu.sync_copy(accfb, accb.at[pl.ds(node0, FSUB)])
    plsc.subcore_barrier()

    def round_body(r, _):
        chunk = 2 * r + cid

        def start_in(blk, p):
            base = sid * EPT + blk * K
            pltpu.async_copy(src.at[pl.ds(base, K)], srcbs[p], isem)
            pltpu.async_copy(dst.at[pl.ds(base, K)], dstbs[p], isem)

        def wait_in(blk, p):
            base = sid * EPT + blk * K
            pltpu.make_async_copy(src.at[pl.ds(base, K)], srcbs[p],
                                  isem).wait()
            pltpu.make_async_copy(dst.at[pl.ds(base, K)], dstbs[p],
                                  isem).wait()

        def mkidx(blk, p):
            ebase = (sid * EPT + blk * K) * NCHUNK + chunk

            def go(i, _):
                s = srcbs[p][pl.ds(i * L, L)]
                d = dstbs[p][pl.ds(i * L, L)]
                src2bs[p][pl.ds(i * L, L)] = s * NCHUNK + chunk
                dst2bs[p][pl.ds(i * L, L)] = d * NCHUNK + chunk
                eidxs[p][pl.ds(i * L, L)] = (
                    lax.iota(jnp.int32, L) * NCHUNK + (ebase + i * (L * NCHUNK)))
                srcSs[p][pl.ds(i * L, L)] = s
                dstSs[p][pl.ds(i * L, L)] = d
                return 0
            lax.fori_loop(0, K // L, go, 0)

        def start_g(p):
            pltpu.async_copy(h_all.at[src2bs[p]], hsbs[p], g1)
            pltpu.async_copy(h_all.at[dst2bs[p]], hdbs[p], g2)
            pltpu.async_copy(e_all.at[eidxs[p]], ebs[p], g3)

        def wait_g(p):
            pltpu.make_async_copy(h_all.at[src2bs[p]], hsbs[p], g1).wait()
            pltpu.make_async_copy(h_all.at[dst2bs[p]], hdbs[p], g2).wait()
            pltpu.make_async_copy(e_all.at[eidxs[p]], ebs[p], g3).wait()

        def wait_scat(p):
            pltpu.make_async_copy(updf, accf.at[dstSs[p]], s1).wait()
            pltpu.make_async_copy(updb, accb.at[srcSs[p]], s2).wait()

        def body(b, p, warm, nxt1, nxt2):
            if nxt1:
                wait_in(b + 1, p ^ 1)
            if warm:
                wait_scat(p ^ 1)
            if nxt1:
                mkidx(b + 1, p ^ 1)
            if nxt2:
                start_in(b + 2, p)
            if nxt1:
                start_g(p ^ 1)
            wait_g(p)

            hsb = hsbs[p]
            hdb = hdbs[p]
            eb = ebs[p]

            def compute(i, _):
                for u in range(UNROLL):
                    j = i * UNROLL + u
                    for q in range(CW // L):
                        sl = pl.ds(q * L, L)
                        hs = hsb[j, sl]
                        hd = hdb[j, sl]
                        ev = eb[j, sl]
                        x = hs * 0.1 + ev - hd * 10.0
                        sig = 1.0 / (1.0 + jnp.exp(-x))
                        updf[j, 0, sl] = sig * (hs * 100.0)
                        updf[j, 1, sl] = sig
                        updb[j, 0, sl] = -(sig * hd)
                        updb[j, 1, sl] = sig
                return 0
            lax.fori_loop(0, K // UNROLL, compute, 0)

            pltpu.async_copy(updf, accf.at[dstSs[p]], s1, add=True)
            pltpu.async_copy(updb, accb.at[srcSs[p]], s2, add=True)

        start_in(0, 0)
        wait_in(0, 0)
        mkidx(0, 0)
        start_in(1, 1)
        start_g(0)

        body(0, 0, warm=False, nxt1=True, nxt2=True)
        body(1, 1, warm=True, nxt1=True, nxt2=True)

        def pair(i, _):
            body(2 + 2 * i, 0, warm=True, nxt1=True, nxt2=True)
            body(3 + 2 * i, 1, warm=True, nxt1=True, nxt2=True)
            return 0
        lax.fori_loop(0, 60, pair, 0)

        body(122, 0, warm=True, nxt1=True, nxt2=True)
        body(123, 1, warm=True, nxt1=True, nxt2=False)
        body(124, 0, warm=True, nxt1=False, nxt2=False)
        wait_scat(0)

        plsc.subcore_barrier()

        for k in range(NFS):
            node0 = sid * NPT + k * FSUB
            pltpu.sync_copy(accf.at[pl.ds(node0, FSUB)], accfb)
            pltpu.sync_copy(accb.at[pl.ds(node0, FSUB)], accbb)
            pltpu.sync_copy(h3.at[pl.ds(node0, FSUB), chunk, :], hb)

            def fl(j, _):
                for q in range(CW // L):
                    sl = pl.ds(q * L, L)
                    nf = accfb[j, 0, sl]
                    df = accfb[j, 1, sl]
                    nb = accbb[j, 0, sl]
                    db = accbb[j, 1, sl]
                    hv = hb[j, sl]
                    hb[j, sl] = (hv * 10.0 + nf / (df + 1e-6)
                                 + nb / (db + 1e-6))
                return 0
            lax.fori_loop(0, FSUB, fl, 0)

            pltpu.sync_copy(hb, out.at[pl.ds(node0, FSUB), chunk, :])

            def zf(j, _):
                z = jnp.zeros((L,), jnp.float32)
                for cc in range(2):
                    for q in range(CW // L):
                        accfb[j, cc, pl.ds(q * L, L)] = z
                return 0
            lax.fori_loop(0, FSUB, zf, 0)
            pltpu.sync_copy(accfb, accf.at[pl.ds(node0, FSUB)])
            pltpu.sync_copy(accfb, accb.at[pl.ds(node0, FSUB)])

        plsc.subcore_barrier()
        return 0

    lax.fori_loop(0, NROUND, round_body, 0)


def _make_sc_kernel():
    mesh = plsc.VectorSubcoreMesh(core_axis_name="c", subcore_axis_name="s")
    idx_t = pltpu.VMEM((K,), jnp.int32)
    row_t = pltpu.VMEM((K, CW), jnp.float32)
    upd_t = pltpu.VMEM((K, 2, CW), jnp.float32)
    return functools.partial(
        pl.kernel,
        mesh=mesh,
        compiler_params=pltpu.CompilerParams(use_tc_tiling_on_sc=False),
        out_type=jax.ShapeDtypeStruct((N, NCHUNK, CW), jnp.float32),
        scratch_types=[
            idx_t, idx_t, idx_t, idx_t,
            idx_t, idx_t, idx_t, idx_t, idx_t, idx_t,
            row_t, row_t,
            row_t, row_t, row_t, row_t,
            upd_t, upd_t,
            idx_t, idx_t, idx_t, idx_t,
            pltpu.VMEM((FSUB, 2, CW), jnp.float32),
            pltpu.VMEM((FSUB, 2, CW), jnp.float32),
            pltpu.VMEM((FSUB, CW), jnp.float32),
            pltpu.VMEM_SHARED((N, 2, CW), jnp.float32),
            pltpu.VMEM_SHARED((N, 2, CW), jnp.float32),
            pltpu.SemaphoreType.DMA,
            pltpu.SemaphoreType.DMA,
            pltpu.SemaphoreType.DMA,
            pltpu.SemaphoreType.DMA,
            pltpu.SemaphoreType.DMA,
            pltpu.SemaphoreType.DMA,
        ],
    )(_sc_grid_kernel)


def _epilogue_body(hh_ref, h_ref, out_ref):
    x = hh_ref[...]
    mean = jnp.mean(x, axis=0, keepdims=True)
    xc = x - mean
    var = jnp.mean(xc * xc, axis=0, keepdims=True)
    y = xc * lax.rsqrt(var + 1e-5)
    out_ref[...] = jnp.maximum(y, 0.0) + h_ref[...]


def kernel(h, e, edge_index):
    h = h.astype(jnp.float32)
    e = e.astype(jnp.float32)
    src = edge_index[0].astype(jnp.int32)
    dst = edge_index[1].astype(jnp.int32)

    h_all = h.reshape(N * NCHUNK, CW)
    h3 = h.reshape(N, NCHUNK, CW)
    e_all = e.reshape(E * NCHUNK, CW)

    sc = _make_sc_kernel()
    hh_pre = sc(h_all, h3, e_all, src, dst).reshape(N, D)

    CB = 128
    return pl.pallas_call(
        _epilogue_body,
        grid=(D // CB,),
        in_specs=[
            pl.BlockSpec((N, CB), lambda c: (0, c)),
            pl.BlockSpec((N, CB), lambda c: (0, c)),
        ],
        out_specs=pl.BlockSpec((N, CB), lambda c: (0, c)),
        out_shape=jax.ShapeDtypeStruct((N, D), jnp.float32),
    )(hh_pre, h)


# speedup vs baseline: 2.0130x; 1.0877x over previous
import functools

import jax
import jax.numpy as jnp
from jax import lax
from jax.experimental import pallas as pl
from jax.experimental.pallas import tpu as pltpu
from jax.experimental.pallas import tpu_sc as plsc

N = 10000
E = 160000
D = 256
NCHUNK = 8
CW = 32
NROUND = NCHUNK // 2
K = 80
EPT = E // 16
NBLK = EPT // K
NPT = N // 16
FSUB = 125
NFS = NPT // FSUB
L = 16
UNROLL = 4


def _sc_grid_kernel(h_all, h3, e_all, src, dst, out,
                    srcb0, srcb1, dstb0, dstb1,
                    src2b0, src2b1, dst2b0, dst2b1, eidx0, eidx1,
                    eb0, eb1, hsb0, hsb1, hdb0, hdb1,
                    updf, updb, srcS0, srcS1, dstS0, dstS1,
                    accfb, accbb, hb,
                    accf, accb, isem, g1, g2, g3, s1, s2):
    srcbs = (srcb0, srcb1)
    dstbs = (dstb0, dstb1)
    src2bs = (src2b0, src2b1)
    dst2bs = (dst2b0, dst2b1)
    eidxs = (eidx0, eidx1)
    ebs = (eb0, eb1)
    hsbs = (hsb0, hsb1)
    hdbs = (hdb0, hdb1)
    srcSs = (srcS0, srcS1)
    dstSs = (dstS0, dstS1)

    cid = lax.axis_index("c")
    sid = lax.axis_index("s")

    def z0(j, _):
        z = jnp.zeros((L,), jnp.float32)
        for cc in range(2):
            for q in range(CW // L):
                accfb[j, cc, pl.ds(q * L, L)] = z
        return 0
    lax.fori_loop(0, FSUB, z0, 0)
    for k in range(NFS):
        node0 = sid * NPT + k * FSUB
        pltpu.sync_copy(accfb, accf.at[pl.ds(node0, FSUB)])
        pltpu.sync_copy(accfb, accb.at[pl.ds(node0, FSUB)])
    plsc.subcore_barrier()

    def round_body(r, _):
        chunk = 2 * r + cid

        def start_in(blk, p):
            base = sid * EPT + blk * K
            pltpu.async_copy(src.at[pl.ds(base, K)], srcbs[p], isem)
            pltpu.async_copy(dst.at[pl.ds(base, K)], dstbs[p], isem)

        def wait_in(blk, p):
            base = sid * EPT + blk * K
            pltpu.make_async_copy(src.at[pl.ds(base, K)], srcbs[p],
                                  isem).wait()
            pltpu.make_async_copy(dst.at[pl.ds(base, K)], dstbs[p],
                                  isem).wait()

        def mkidx(blk, p):
            ebase = (sid * EPT + blk * K) * NCHUNK + chunk

            def go(i, _):
                s = srcbs[p][pl.ds(i * L, L)]
                d = dstbs[p][pl.ds(i * L, L)]
                src2bs[p][pl.ds(i * L, L)] = s * NCHUNK + chunk
                dst2bs[p][pl.ds(i * L, L)] = d * NCHUNK + chunk
                eidxs[p][pl.ds(i * L, L)] = (
                    lax.iota(jnp.int32, L) * NCHUNK + (ebase + i * (L * NCHUNK)))
                return 0
            lax.fori_loop(0, K // L, go, 0)

        def mkscatidx(p):
            def go(i, _):
                sl = pl.ds(i * L, L)
                srcSs[p][sl] = srcbs[p][sl]
                dstSs[p][sl] = dstbs[p][sl]
                return 0
            lax.fori_loop(0, K // L, go, 0)

        updfs = (updf, accfb)
        updbs = (updb, accbb)

        def updf_src(p):
            return updfs[p] if p == 0 else updfs[p].at[pl.ds(0, K)]

        def updb_src(p):
            return updbs[p] if p == 0 else updbs[p].at[pl.ds(0, K)]

        def start_g(p):
            pltpu.async_copy(h_all.at[src2bs[p]], hsbs[p], g1)
            pltpu.async_copy(h_all.at[dst2bs[p]], hdbs[p], g2)
            pltpu.async_copy(e_all.at[eidxs[p]], ebs[p], g3)

        def wait_g(p):
            pltpu.make_async_copy(h_all.at[src2bs[p]], hsbs[p], g1).wait()
            pltpu.make_async_copy(h_all.at[dst2bs[p]], hdbs[p], g2).wait()
            pltpu.make_async_copy(e_all.at[eidxs[p]], ebs[p], g3).wait()

        def wait_scat(p):
            pltpu.make_async_copy(updf_src(p), accf.at[dstSs[p]], s1).wait()
            pltpu.make_async_copy(updb_src(p), accb.at[srcSs[p]], s2).wait()

        def body(b, p, warm, nxt1, nxt2):
            if nxt1:
                wait_in(b + 1, p ^ 1)
            if nxt1:
                mkidx(b + 1, p ^ 1)
            if warm:
                wait_scat(p)
            mkscatidx(p)
            if nxt2:
                start_in(b + 2, p)
            if nxt1:
                start_g(p ^ 1)
            wait_g(p)

            hsb = hsbs[p]
            hdb = hdbs[p]
            eb = ebs[p]
            uf = updfs[p]
            ub = updbs[p]

            def compute(i, _):
                for u in range(UNROLL):
                    j = i * UNROLL + u
                    for q in range(CW // L):
                        sl = pl.ds(q * L, L)
                        hs = hsb[j, sl]
                        hd = hdb[j, sl]
                        ev = eb[j, sl]
                        x = hs * 0.1 + ev - hd * 10.0
                        sig = 1.0 / (1.0 + jnp.exp(-x))
                        uf[j, 0, sl] = sig * (hs * 100.0)
                        uf[j, 1, sl] = sig
                        ub[j, 0, sl] = -(sig * hd)
                        ub[j, 1, sl] = sig
                return 0
            lax.fori_loop(0, K // UNROLL, compute, 0)

            pltpu.async_copy(updf_src(p), accf.at[dstSs[p]], s1, add=True)
            pltpu.async_copy(updb_src(p), accb.at[srcSs[p]], s2, add=True)

        start_in(0, 0)
        wait_in(0, 0)
        mkidx(0, 0)
        start_in(1, 1)
        start_g(0)

        body(0, 0, warm=False, nxt1=True, nxt2=True)
        body(1, 1, warm=False, nxt1=True, nxt2=True)

        def pair(i, _):
            body(2 + 2 * i, 0, warm=True, nxt1=True, nxt2=True)
            body(3 + 2 * i, 1, warm=True, nxt1=True, nxt2=True)
            return 0
        lax.fori_loop(0, 60, pair, 0)

        body(122, 0, warm=True, nxt1=True, nxt2=True)
        body(123, 1, warm=True, nxt1=True, nxt2=False)
        body(124, 0, warm=True, nxt1=False, nxt2=False)
        wait_scat(1)
        wait_scat(0)

        plsc.subcore_barrier()

        for k in range(NFS):
            node0 = sid * NPT + k * FSUB
            pltpu.sync_copy(accf.at[pl.ds(node0, FSUB)], accfb)
            pltpu.sync_copy(accb.at[pl.ds(node0, FSUB)], accbb)
            pltpu.sync_copy(h3.at[pl.ds(node0, FSUB), chunk, :], hb)

            def fl(j, _):
                for q in range(CW // L):
                    sl = pl.ds(q * L, L)
                    nf = accfb[j, 0, sl]
                    df = accfb[j, 1, sl]
                    nb = accbb[j, 0, sl]
                    db = accbb[j, 1, sl]
                    hv = hb[j, sl]
                    hb[j, sl] = (hv * 10.0 + nf / (df + 1e-6)
                                 + nb / (db + 1e-6))
                return 0
            lax.fori_loop(0, FSUB, fl, 0)

            pltpu.sync_copy(hb, out.at[pl.ds(node0, FSUB), chunk, :])

            def zf(j, _):
                z = jnp.zeros((L,), jnp.float32)
                for cc in range(2):
                    for q in range(CW // L):
                        accfb[j, cc, pl.ds(q * L, L)] = z
                return 0
            lax.fori_loop(0, FSUB, zf, 0)
            pltpu.sync_copy(accfb, accf.at[pl.ds(node0, FSUB)])
            pltpu.sync_copy(accfb, accb.at[pl.ds(node0, FSUB)])

        plsc.subcore_barrier()
        return 0

    lax.fori_loop(0, NROUND, round_body, 0)


def _make_sc_kernel():
    mesh = plsc.VectorSubcoreMesh(core_axis_name="c", subcore_axis_name="s")
    idx_t = pltpu.VMEM((K,), jnp.int32)
    row_t = pltpu.VMEM((K, CW), jnp.float32)
    upd_t = pltpu.VMEM((K, 2, CW), jnp.float32)
    return functools.partial(
        pl.kernel,
        mesh=mesh,
        compiler_params=pltpu.CompilerParams(use_tc_tiling_on_sc=False),
        out_type=jax.ShapeDtypeStruct((N, NCHUNK, CW), jnp.float32),
        scratch_types=[
            idx_t, idx_t, idx_t, idx_t,
            idx_t, idx_t, idx_t, idx_t, idx_t, idx_t,
            row_t, row_t,
            row_t, row_t, row_t, row_t,
            upd_t, upd_t,
            idx_t, idx_t, idx_t, idx_t,
            pltpu.VMEM((FSUB, 2, CW), jnp.float32),
            pltpu.VMEM((FSUB, 2, CW), jnp.float32),
            pltpu.VMEM((FSUB, CW), jnp.float32),
            pltpu.VMEM_SHARED((N, 2, CW), jnp.float32),
            pltpu.VMEM_SHARED((N, 2, CW), jnp.float32),
            pltpu.SemaphoreType.DMA,
            pltpu.SemaphoreType.DMA,
            pltpu.SemaphoreType.DMA,
            pltpu.SemaphoreType.DMA,
            pltpu.SemaphoreType.DMA,
            pltpu.SemaphoreType.DMA,
        ],
    )(_sc_grid_kernel)


def _epilogue_body(hh_ref, h_ref, out_ref):
    x = hh_ref[...]
    mean = jnp.mean(x, axis=0, keepdims=True)
    xc = x - mean
    var = jnp.mean(xc * xc, axis=0, keepdims=True)
    y = xc * lax.rsqrt(var + 1e-5)
    out_ref[...] = jnp.maximum(y, 0.0) + h_ref[...]


def kernel(h, e, edge_index):
    h = h.astype(jnp.float32)
    e = e.astype(jnp.float32)
    src = edge_index[0].astype(jnp.int32)
    dst = edge_index[1].astype(jnp.int32)

    h_all = h.reshape(N * NCHUNK, CW)
    h3 = h.reshape(N, NCHUNK, CW)
    e_all = e.reshape(E * NCHUNK, CW)

    sc = _make_sc_kernel()
    hh_pre = sc(h_all, h3, e_all, src, dst).reshape(N, D)

    CB = 128
    return pl.pallas_call(
        _epilogue_body,
        grid=(D // CB,),
        in_specs=[
            pl.BlockSpec((N, CB), lambda c: (0, c)),
            pl.BlockSpec((N, CB), lambda c: (0, c)),
        ],
        out_specs=pl.BlockSpec((N, CB), lambda c: (0, c)),
        out_shape=jax.ShapeDtypeStruct((N, D), jnp.float32),
    )(hh_pre, h)
